# baseline (speedup 1.0000x reference)
import functools

import jax
import jax.numpy as jnp
from jax.experimental import pallas as pl
from jax.experimental.pallas import tpu as pltpu

_C = 256
_G = 128


def _scan_body(x_ref, m_ref, o_ref, carry_ref, *, c, g):
    j = pl.program_id(1)

    @pl.when(j == 0)
    def _():
        carry_ref[...] = jnp.zeros_like(carry_ref)

    p = x_ref[...] * m_ref[...]
    d = p.shape[1]
    row = jax.lax.broadcasted_iota(jnp.int32, (g, g), 0)
    col = jax.lax.broadcasted_iota(jnp.int32, (g, g), 1)
    tri = (row >= col).astype(jnp.float32)

    carry = carry_ref[...]
    for grp in range(c // g):
        pg = p[grp * g:(grp + 1) * g, :]
        og = jax.lax.dot(
            tri, pg,
            precision=jax.lax.Precision.DEFAULT,
            preferred_element_type=jnp.float32,
        ) + carry
        o_ref[grp * g:(grp + 1) * g, :] = og
        carry = carry + jnp.sum(pg, axis=0, keepdims=True)
    carry_ref[...] = carry


def kernel(x, mask):
    b, n, d = x.shape
    c = _C
    grid = (b, n // c)
    spec = pl.BlockSpec((None, c, d), lambda bi, ji: (bi, ji, 0))
    return pl.pallas_call(
        functools.partial(_scan_body, c=c, g=_G),
        grid=grid,
        in_specs=[spec, spec],
        out_specs=spec,
        out_shape=jax.ShapeDtypeStruct((b, n, d), x.dtype),
        scratch_shapes=[pltpu.VMEM((1, d), jnp.float32)],
        compiler_params=pltpu.CompilerParams(
            dimension_semantics=("arbitrary", "arbitrary"),
        ),
    )(x, mask)


# speedup vs baseline: 1.0421x; 1.0421x over previous
import functools

import jax
import jax.numpy as jnp
from jax.experimental import pallas as pl
from jax.experimental.pallas import tpu as pltpu

_C = 1024
_G = 128


def _scan_body(x_ref, m_ref, o_ref, carry_ref, *, c, g):
    j = pl.program_id(1)

    @pl.when(j == 0)
    def _():
        carry_ref[...] = jnp.zeros_like(carry_ref)

    p = x_ref[...] * m_ref[...]
    d = p.shape[1]
    row = jax.lax.broadcasted_iota(jnp.int32, (g, g), 0)
    col = jax.lax.broadcasted_iota(jnp.int32, (g, g), 1)
    tri = (row >= col).astype(jnp.float32)

    carry = carry_ref[...]
    for grp in range(c // g):
        pg = p[grp * g:(grp + 1) * g, :]
        og = jax.lax.dot(
            tri, pg,
            precision=jax.lax.Precision.DEFAULT,
            preferred_element_type=jnp.float32,
        ) + carry
        o_ref[grp * g:(grp + 1) * g, :] = og
        carry = carry + jnp.sum(pg, axis=0, keepdims=True)
    carry_ref[...] = carry


def kernel(x, mask):
    b, n, d = x.shape
    c = _C
    grid = (b, n // c)
    spec = pl.BlockSpec((None, c, d), lambda bi, ji: (bi, ji, 0))
    return pl.pallas_call(
        functools.partial(_scan_body, c=c, g=_G),
        grid=grid,
        in_specs=[spec, spec],
        out_specs=spec,
        out_shape=jax.ShapeDtypeStruct((b, n, d), x.dtype),
        scratch_shapes=[pltpu.VMEM((1, d), jnp.float32)],
        compiler_params=pltpu.CompilerParams(
            dimension_semantics=("arbitrary", "arbitrary"),
        ),
    )(x, mask)
